# baseline (speedup 1.0000x reference)
import functools

import jax
import jax.numpy as jnp
from jax import lax
from jax.experimental import pallas as pl
from jax.experimental.pallas import tpu as pltpu
from jax.experimental.pallas import tpu_sc as plsc

_TPS = 2


def kernel(x):
    B, F, K, ED = x.shape
    G = F // _TPS
    NC, NS = 2, 16
    NW = NC * NS
    pairs = B * G
    per_w = pairs // NW

    xt = jnp.swapaxes(x, 2, 3)

    mesh = plsc.VectorSubcoreMesh(core_axis_name="c", subcore_axis_name="s")

    NBUF = 4
    HED = ED // 2
    n_chunks = _TPS * per_w * 2

    @functools.partial(
        pl.kernel,
        out_type=jax.ShapeDtypeStruct((B, G, _TPS * ED, K), jnp.float32),
        mesh=mesh,
        scratch_types=[
            [pltpu.VMEM((HED, K), jnp.float32) for _ in range(NBUF)],
            [pltpu.SemaphoreType.DMA for _ in range(NBUF)],
            [pltpu.SemaphoreType.DMA for _ in range(NBUF)],
        ],
    )
    def merge(xt_hbm, outt_hbm, bufs, isems, osems):
        wid = lax.axis_index("s") * NC + lax.axis_index("c")
        base = wid * per_w

        def src(j):
            q = base + j // 4
            i = (j // 2) % _TPS
            h = j % 2
            b = q // G
            g = q % G
            return xt_hbm.at[b, _TPS * g + i, pl.ds(h * HED, HED), :]

        def dst(j):
            q = base + j // 4
            i = (j // 2) % _TPS
            h = j % 2
            b = q // G
            g = q % G
            return outt_hbm.at[b, g, pl.ds(i * ED + h * HED, HED), :]

        ins = [None] * NBUF
        outs = [None] * NBUF
        for j in range(min(2, n_chunks)):
            ins[j % NBUF] = pltpu.async_copy(src(j), bufs[j % NBUF], isems[j % NBUF])
        for j in range(n_chunks):
            s = j % NBUF
            ins[s].wait()
            if j + 2 < n_chunks:
                s2 = (j + 2) % NBUF
                if outs[s2] is not None:
                    outs[s2].wait()
                ins[s2] = pltpu.async_copy(src(j + 2), bufs[s2], isems[s2])
            outs[s] = pltpu.async_copy(bufs[s], dst(j), osems[s])
        for o in outs:
            if o is not None:
                o.wait()

    outt = merge(xt)
    return jnp.swapaxes(outt, 2, 3)
